# Initial kernel scaffold; baseline (speedup 1.0000x reference)
import jax
import jax.numpy as jnp
from jax.experimental import pallas as pl


def kernel(inputs, table):
    raise NotImplementedError("write your pallas kernel here")


# speedup vs baseline: 1.3089x; 1.3089x over previous
import functools

import jax
import jax.numpy as jnp
from jax import lax
from jax.experimental import pallas as pl
from jax.experimental.pallas import tpu as pltpu
from jax.experimental.pallas import tpu_sc as plsc

BATCH = 4096
HIST = 200
EMBED = 32

NC = 2
NS = 16
NW = NC * NS

B = BATCH * HIST
CH = 128
ROWS_PER_W = B // NW
NCH = ROWS_PER_W // CH


def _gather_body(table_hbm, idx_hbm, out_hbm, idx_v, rows_v, sem):
    wid = lax.axis_index("s") * NC + lax.axis_index("c")
    chunk_base = wid * NCH
    row_base = wid * ROWS_PER_W
    pltpu.sync_copy(idx_hbm.at[pl.ds(chunk_base, NCH)], idx_v)

    @pl.loop(0, NCH)
    def _(c):
        pltpu.async_copy(table_hbm.at[idx_v.at[c]], rows_v, sem).wait()
        pltpu.sync_copy(rows_v, out_hbm.at[pl.ds(row_base + c * CH, CH)])


@jax.jit
def _gather(table, idx2d):
    mesh = plsc.VectorSubcoreMesh(
        core_axis_name="c", subcore_axis_name="s",
        num_cores=NC, num_subcores=NS,
    )
    f = pl.kernel(
        _gather_body,
        out_type=jax.ShapeDtypeStruct((B, EMBED), jnp.float32),
        mesh=mesh,
        scratch_types=[
            pltpu.VMEM((NCH, CH), jnp.int32),
            pltpu.VMEM((CH, EMBED), jnp.float32),
            pltpu.SemaphoreType.DMA,
        ],
        compiler_params=pltpu.CompilerParams(use_tc_tiling_on_sc=False),
    )
    return f(table, idx2d)


def kernel(inputs, table):
    idx2d = inputs.reshape(B // CH, CH)
    out = _gather(table, idx2d)
    return out.reshape(BATCH, HIST, EMBED)


# speedup vs baseline: 1.4990x; 1.1453x over previous
import jax
import jax.numpy as jnp
from jax import lax
from jax.experimental import pallas as pl
from jax.experimental.pallas import tpu as pltpu
from jax.experimental.pallas import tpu_sc as plsc

BATCH = 4096
HIST = 200
EMBED = 32

NC = 2
NS = 16
NW = NC * NS

B = BATCH * HIST
CH = 128
ROWS_PER_W = B // NW
NCH = ROWS_PER_W // CH
K = 10
NT = NCH // K
BLOCK_ROWS = K * CH


def _gather_body(table_hbm, idx_hbm, out_hbm, idx_v, rows0, rows1, g0, g1,
                 s0, s1):
    wid = lax.axis_index("s") * NC + lax.axis_index("c")
    chunk_base = wid * NCH
    row_base = wid * ROWS_PER_W
    pltpu.sync_copy(idx_hbm.at[pl.ds(chunk_base, NCH)], idx_v)

    def fire_block(t, rows_v, g_sem):
        descs = []
        for j in range(K):
            descs.append(pltpu.async_copy(
                table_hbm.at[idx_v.at[t * K + j]],
                rows_v.at[pl.ds(j * CH, CH)], g_sem))
        for d in descs:
            d.wait()

    def store_block(t, rows_v, s_sem):
        pltpu.async_copy(
            rows_v, out_hbm.at[pl.ds(row_base + t * BLOCK_ROWS, BLOCK_ROWS)],
            s_sem)

    def wait_store(rows_v, s_sem):
        pltpu.make_async_copy(
            rows_v, out_hbm.at[pl.ds(row_base, BLOCK_ROWS)], s_sem).wait()

    @pl.loop(0, NT, step=2)
    def _(tp):
        @pl.when(tp >= 2)
        def _():
            wait_store(rows0, s0)
        fire_block(tp, rows0, g0)
        store_block(tp, rows0, s0)

        @pl.when(tp >= 2)
        def _():
            wait_store(rows1, s1)
        fire_block(tp + 1, rows1, g1)
        store_block(tp + 1, rows1, s1)

    wait_store(rows0, s0)
    wait_store(rows1, s1)


@jax.jit
def _gather(table, idx2d):
    mesh = plsc.VectorSubcoreMesh(
        core_axis_name="c", subcore_axis_name="s",
        num_cores=NC, num_subcores=NS,
    )
    f = pl.kernel(
        _gather_body,
        out_type=jax.ShapeDtypeStruct((B, EMBED), jnp.float32),
        mesh=mesh,
        scratch_types=[
            pltpu.VMEM((NCH, CH), jnp.int32),
            pltpu.VMEM((BLOCK_ROWS, EMBED), jnp.float32),
            pltpu.VMEM((BLOCK_ROWS, EMBED), jnp.float32),
            pltpu.SemaphoreType.DMA,
            pltpu.SemaphoreType.DMA,
            pltpu.SemaphoreType.DMA,
            pltpu.SemaphoreType.DMA,
        ],
        compiler_params=pltpu.CompilerParams(use_tc_tiling_on_sc=False),
    )
    return f(table, idx2d)


def kernel(inputs, table):
    idx2d = inputs.reshape(B // CH, CH)
    out = _gather(table, idx2d)
    return out.reshape(BATCH, HIST, EMBED)
